# baseline (speedup 1.0000x reference)
import functools

import jax
import jax.numpy as jnp
from jax import lax
from jax.experimental import pallas as pl
from jax.experimental.pallas import tpu as pltpu
from jax.experimental.pallas import tpu_sc as plsc

_NC = 2
_NS = 16
_NW = _NC * _NS
_C = 32
_K = 3
_A = 2
_BSC = 1
_TB = 2048


def _sc_body(tok_hbm, pos_hbm, out_hbm, *scratch, NB, T, D):
    bufs, sems = scratch[:_K + 2], scratch[_K + 2:]
    tok, pos = list(bufs[:_K]), list(bufs[_K:])
    sin, sout, spos = list(sems[:_K]), list(sems[_K:2 * _K]), list(sems[2 * _K:])
    span = T // _NW
    n_chunks = span // _C
    n_steps = n_chunks * NB
    wid = lax.axis_index("s") * _NC + lax.axis_index("c")
    t0 = wid * span

    din, dout, dpos = {}, {}, {}

    def start_in(s):
        tc, b = divmod(s, NB)
        tt = t0 + tc * _C
        din[s] = pltpu.async_copy(
            tok_hbm.at[b, pl.ds(tt, _C)], tok[s % _K], sin[s % _K])

    def start_out(s):
        tc, b = divmod(s, NB)
        tt = t0 + tc * _C
        dout[s] = pltpu.async_copy(
            tok[s % _K], out_hbm.at[b, pl.ds(tt, _C)], sout[s % _K])

    def start_pos(tc):
        dpos[tc] = pltpu.async_copy(
            pos_hbm.at[pl.ds(t0 + tc * _C, _C)], pos[tc % 2], spos[tc % 2])

    start_pos(0)
    for s in range(_A):
        start_in(s)

    for s in range(n_steps):
        tc, b = divmod(s, NB)
        if b == 0:
            dpos[tc].wait()
            if tc + 1 < n_chunks:
                start_pos(tc + 1)
        din[s].wait()

        tbuf, pbuf = tok[s % _K], pos[tc % 2]

        def row(r, _, tbuf=tbuf, pbuf=pbuf):
            def add16(i, _=None):
                sl = pl.ds(i * 16, 16)
                plsc.addupdate(tbuf.at[r, sl], pbuf[r, sl])

            plsc.parallel_loop(0, D // 16, 1, unroll=8)(add16)
            return _

        lax.fori_loop(0, _C, row, 0)
        start_out(s)
        if s + _A < n_steps:
            if s - _A >= 0:
                dout[s - _A].wait()
            start_in(s + _A)

    for s in range(n_steps - 2 * _A, n_steps):
        if s >= 0:
            dout[s].wait()


def _tc_add(tok_ref, pos_ref, out_ref):
    out_ref[...] = tok_ref[...] + pos_ref[...]


def kernel(tokens, pos_table):
    B, T, D = tokens.shape

    sc_add = pl.kernel(
        functools.partial(_sc_body, NB=_BSC, T=T, D=D),
        out_type=jax.ShapeDtypeStruct((_BSC, T, D), jnp.float32),
        mesh=plsc.VectorSubcoreMesh(core_axis_name="c", subcore_axis_name="s"),
        scratch_types=(
            [pltpu.VMEM((_C, D), jnp.float32)] * (_K + 2)
            + [pltpu.SemaphoreType.DMA] * (2 * _K + 2)
        ),
    )
    sc_out = sc_add(tokens, pos_table)

    n_tc = B - _BSC
    tc_out = pl.pallas_call(
        _tc_add,
        grid=(T // _TB, n_tc),
        in_specs=[
            pl.BlockSpec((1, _TB, D), lambda t, b: (b + _BSC, t, 0)),
            pl.BlockSpec((_TB, D), lambda t, b: (t, 0)),
        ],
        out_specs=pl.BlockSpec((1, _TB, D), lambda t, b: (b, t, 0)),
        out_shape=jax.ShapeDtypeStruct((n_tc, T, D), tokens.dtype),
    )(tokens, pos_table)

    return jnp.concatenate([sc_out, tc_out], axis=0)


# speedup vs baseline: 1.4772x; 1.4772x over previous
import functools

import jax
import jax.numpy as jnp
from jax import lax
from jax.experimental import pallas as pl
from jax.experimental.pallas import tpu as pltpu
from jax.experimental.pallas import tpu_sc as plsc

_NC = 2
_NS = 16
_NW = _NC * _NS
_C = 16
_K = 6
_A = 3


def _sc_body(tok_hbm, pos_hbm, out_hbm, *scratch, B, T, D):
    bufs, sems = scratch[:_K + 2], scratch[_K + 2:]
    tok, pos = list(bufs[:_K]), list(bufs[_K:])
    sin, sout, spos = list(sems[:_K]), list(sems[_K:2 * _K]), list(sems[2 * _K:])
    span = T // _NW
    n_chunks = span // _C
    n_steps = n_chunks * B
    wid = lax.axis_index("s") * _NC + lax.axis_index("c")
    t0 = wid * span

    din, dout, dpos = {}, {}, {}

    def start_in(s):
        tc, b = divmod(s, B)
        tt = t0 + tc * _C
        din[s] = pltpu.async_copy(
            tok_hbm.at[b, pl.ds(tt, _C)], tok[s % _K], sin[s % _K])

    def start_out(s):
        tc, b = divmod(s, B)
        tt = t0 + tc * _C
        dout[s] = pltpu.async_copy(
            tok[s % _K], out_hbm.at[b, pl.ds(tt, _C)], sout[s % _K])

    def start_pos(tc):
        dpos[tc] = pltpu.async_copy(
            pos_hbm.at[pl.ds(t0 + tc * _C, _C)], pos[tc % 2], spos[tc % 2])

    start_pos(0)
    if n_chunks > 1:
        start_pos(1)
    for s in range(_A):
        start_in(s)

    for s in range(n_steps):
        tc, b = divmod(s, B)
        if b == 0:
            dpos[tc].wait()
            if tc + 2 < n_chunks:
                start_pos(tc + 2)
        din[s].wait()

        tbuf, pbuf = tok[s % _K], pos[tc % 2]

        def row(r, _, tbuf=tbuf, pbuf=pbuf):
            def add16(i, _=None):
                sl = pl.ds(i * 16, 16)
                plsc.addupdate(tbuf.at[r, sl], pbuf[r, sl])

            plsc.parallel_loop(0, D // 16, 1, unroll=8)(add16)
            return _

        lax.fori_loop(0, _C, row, 0)
        start_out(s)
        if s + _A < n_steps:
            if s + _A - _K >= 0:
                dout[s + _A - _K].wait()
            start_in(s + _A)

    for s in range(max(0, n_steps - _K), n_steps):
        dout[s].wait()


def kernel(tokens, pos_table):
    B, T, D = tokens.shape

    sc_add = pl.kernel(
        functools.partial(_sc_body, B=B, T=T, D=D),
        out_type=jax.ShapeDtypeStruct((B, T, D), jnp.float32),
        mesh=plsc.VectorSubcoreMesh(core_axis_name="c", subcore_axis_name="s"),
        scratch_types=(
            [pltpu.VMEM((_C, D), jnp.float32)] * (_K + 2)
            + [pltpu.SemaphoreType.DMA] * (2 * _K + 2)
        ),
    )

    return sc_add(tokens, pos_table)


# speedup vs baseline: 1.4815x; 1.0029x over previous
import functools

import jax
import jax.numpy as jnp
from jax import lax
from jax.experimental import pallas as pl
from jax.experimental.pallas import tpu as pltpu
from jax.experimental.pallas import tpu_sc as plsc

_NC = 2
_NS = 16
_NW = _NC * _NS
_C = 16
_K = 6
_A = 3


def _sc_body(tok_hbm, pos_hbm, out_hbm, *scratch, B, T, D):
    bufs, sems = scratch[:_K + 2], scratch[_K + 2:]
    tok, pos = list(bufs[:_K]), list(bufs[_K:])
    sin, sout, spos = list(sems[:_K]), list(sems[_K:2 * _K]), list(sems[2 * _K:])
    span = T // _NW
    n_chunks = span // _C
    n_steps = n_chunks * B
    wid = lax.axis_index("s") * _NC + lax.axis_index("c")
    t0 = wid * span

    din, dout, dpos = {}, {}, {}

    def start_in(s):
        tc, b = divmod(s, B)
        tt = t0 + tc * _C
        din[s] = pltpu.async_copy(
            tok_hbm.at[b, pl.ds(tt, _C)], tok[s % _K], sin[s % _K])

    def start_out(s):
        tc, b = divmod(s, B)
        tt = t0 + tc * _C
        dout[s] = pltpu.async_copy(
            tok[s % _K], out_hbm.at[b, pl.ds(tt, _C)], sout[s % _K])

    def start_pos(tc):
        dpos[tc] = pltpu.async_copy(
            pos_hbm.at[pl.ds(t0 + tc * _C, _C)], pos[tc % 2], spos[tc % 2])

    start_pos(0)
    for s in range(_A):
        start_in(s)

    for s in range(n_steps):
        tc, b = divmod(s, B)
        if b == 0:
            dpos[tc].wait()
            if tc + 1 < n_chunks:
                start_pos(tc + 1)
        din[s].wait()

        tbuf, pbuf = tok[s % _K], pos[tc % 2]

        def row(r, _, tbuf=tbuf, pbuf=pbuf):
            def add16(i, _=None):
                sl = pl.ds(i * 16, 16)
                plsc.addupdate(tbuf.at[r, sl], pbuf[r, sl])

            plsc.parallel_loop(0, D // 16, 1, unroll=8)(add16)
            return _

        lax.fori_loop(0, _C, row, 0)
        start_out(s)
        if s + _A < n_steps:
            if s + _A - _K >= 0:
                dout[s + _A - _K].wait()
            start_in(s + _A)

    for s in range(max(0, n_steps - _K), n_steps):
        dout[s].wait()


def kernel(tokens, pos_table):
    B, T, D = tokens.shape

    sc_add = pl.kernel(
        functools.partial(_sc_body, B=B, T=T, D=D),
        out_type=jax.ShapeDtypeStruct((B, T, D), jnp.float32),
        mesh=plsc.VectorSubcoreMesh(core_axis_name="c", subcore_axis_name="s"),
        scratch_types=(
            [pltpu.VMEM((_C, D), jnp.float32)] * (_K + 2)
            + [pltpu.SemaphoreType.DMA] * (2 * _K + 2)
        ),
    )

    return sc_add(tokens, pos_table)


# speedup vs baseline: 1.4900x; 1.0057x over previous
import functools

import jax
import jax.numpy as jnp
from jax import lax
from jax.experimental import pallas as pl
from jax.experimental.pallas import tpu as pltpu
from jax.experimental.pallas import tpu_sc as plsc

_NC = 2
_NS = 16
_NW = _NC * _NS
_C = 16
_K = 8
_A = 4


def _sc_body(tok_hbm, pos_hbm, out_hbm, *scratch, B, T, D):
    bufs, sems = scratch[:_K + 2], scratch[_K + 2:]
    tok, pos = list(bufs[:_K]), list(bufs[_K:])
    sin, sout, spos = list(sems[:_K]), list(sems[_K:2 * _K]), list(sems[2 * _K:])
    span = T // _NW
    n_chunks = span // _C
    n_steps = n_chunks * B
    wid = lax.axis_index("s") * _NC + lax.axis_index("c")
    t0 = wid * span

    din, dout, dpos = {}, {}, {}

    def start_in(s):
        tc, b = divmod(s, B)
        tt = t0 + tc * _C
        din[s] = pltpu.async_copy(
            tok_hbm.at[b, pl.ds(tt, _C)], tok[s % _K], sin[s % _K])

    def start_out(s):
        tc, b = divmod(s, B)
        tt = t0 + tc * _C
        dout[s] = pltpu.async_copy(
            tok[s % _K], out_hbm.at[b, pl.ds(tt, _C)], sout[s % _K])

    def start_pos(tc):
        dpos[tc] = pltpu.async_copy(
            pos_hbm.at[pl.ds(t0 + tc * _C, _C)], pos[tc % 2], spos[tc % 2])

    start_pos(0)
    for s in range(_A):
        start_in(s)

    for s in range(n_steps):
        tc, b = divmod(s, B)
        if b == 0:
            dpos[tc].wait()
            if tc + 1 < n_chunks:
                start_pos(tc + 1)
        din[s].wait()

        tbuf, pbuf = tok[s % _K], pos[tc % 2]

        def row(r, _, tbuf=tbuf, pbuf=pbuf):
            def add16(i, _=None):
                sl = pl.ds(i * 16, 16)
                plsc.addupdate(tbuf.at[r, sl], pbuf[r, sl])

            plsc.parallel_loop(0, D // 16, 1, unroll=8)(add16)
            return _

        lax.fori_loop(0, _C, row, 0)
        start_out(s)
        if s + _A < n_steps:
            if s + _A - _K >= 0:
                dout[s + _A - _K].wait()
            start_in(s + _A)

    for s in range(max(0, n_steps - _K), n_steps):
        dout[s].wait()


def kernel(tokens, pos_table):
    B, T, D = tokens.shape

    sc_add = pl.kernel(
        functools.partial(_sc_body, B=B, T=T, D=D),
        out_type=jax.ShapeDtypeStruct((B, T, D), jnp.float32),
        mesh=plsc.VectorSubcoreMesh(core_axis_name="c", subcore_axis_name="s"),
        scratch_types=(
            [pltpu.VMEM((_C, D), jnp.float32)] * (_K + 2)
            + [pltpu.SemaphoreType.DMA] * (2 * _K + 2)
        ),
    )

    return sc_add(tokens, pos_table)
